# Initial kernel scaffold; baseline (speedup 1.0000x reference)
import jax
import jax.numpy as jnp
from jax.experimental import pallas as pl


def kernel(x_s, x_t, edge_index, emb, W1, b1, W2, b2, Wl, bl, L1W, L1b, L2W, L2b, L3W, L3b):
    raise NotImplementedError("write your pallas kernel here")


# speedup vs baseline: 11.8342x; 11.8342x over previous
import jax
import jax.numpy as jnp
import numpy as np
from jax.experimental import pallas as pl
from jax.experimental.pallas import tpu as pltpu

N = 10000
E = 30000
ECHUNK = 1000
NEC = E // ECHUNK
B = 1000
NB = N // B
F1 = 512
F2 = 720
_INTERPRET = False


def _deg_kernel(col_ref, dinv_ref, deg_ref):
    step = pl.program_id(0)

    @pl.when(step == 0)
    def _init():
        deg_ref[...] = jnp.ones_like(deg_ref)

    def body(i, carry):
        c = col_ref[0, 0, i]
        deg_ref[pl.ds(c, 1), :] = deg_ref[pl.ds(c, 1), :] + 1.0
        return carry

    jax.lax.fori_loop(0, ECHUNK, body, 0)

    @pl.when(step == NEC - 1)
    def _fin():
        dinv_ref[...] = jax.lax.rsqrt(deg_ref[...])


def _emb_kernel(idxe_ref, dinv_ref, g2_ref, xs_ref):
    lane = jax.lax.broadcasted_iota(jnp.int32, (B, 45 * 21), 1) % 21
    u = (idxe_ref[...] == lane).astype(jnp.float32)
    x = jnp.dot(u, g2_ref[...], preferred_element_type=jnp.float32)
    xs_ref[...] = x * dinv_ref[...]


def _spmm_kernel(row_ref, col_ref, xs_ref, acc_ref):
    step = pl.program_id(0)

    @pl.when(step == 0)
    def _init():
        acc_ref[...] = xs_ref[...]

    def body(i, carry):
        r = row_ref[0, 0, i]
        c = col_ref[0, 0, i]
        acc_ref[pl.ds(c, 1), :] = acc_ref[pl.ds(c, 1), :] + xs_ref[pl.ds(r, 1), :]
        return carry

    jax.lax.fori_loop(0, ECHUNK, body, 0)


def _proj_kernel(acc_ref, dinv_ref, w1_ref, b1_ref, w2_ref, p_ref):
    d = dinv_ref[...]
    h = jnp.dot(acc_ref[...] * d, w1_ref[...],
                preferred_element_type=jnp.float32) + b1_ref[...]
    h = jnp.maximum(h, 0.0)
    p = jnp.dot(h, w2_ref[...], preferred_element_type=jnp.float32)
    p_ref[...] = p * d


def _mlp_kernel(acc_ref, dinv_ref, b2_ref, wl_ref, bl_ref, l1w_ref, l1b_ref,
                l2w_ref, l2b_ref, l3w_ref, l3b_ref, out_ref):
    flat = acc_ref[...] * dinv_ref[...] + b2_ref[...]
    o = jnp.dot(flat, wl_ref[...], preferred_element_type=jnp.float32) + bl_ref[...]
    o = jnp.where(o >= 0.0, o, 0.01 * o)
    o = jnp.maximum(jnp.dot(o, l1w_ref[...],
                            preferred_element_type=jnp.float32) + l1b_ref[...], 0.0)
    o = jnp.maximum(jnp.dot(o, l2w_ref[...],
                            preferred_element_type=jnp.float32) + l2b_ref[...], 0.0)
    z = jnp.dot(o, l3w_ref[...], preferred_element_type=jnp.float32) + l3b_ref[...]
    out_ref[...] = 1.0 / (1.0 + jnp.exp(-z))


def _smem_spec(i_map):
    return pl.BlockSpec((1, 1, ECHUNK), i_map, memory_space=pltpu.SMEM)


def _whole(shape):
    return pl.BlockSpec(shape, lambda i: (0,) * len(shape))


def _spmm(row3, col3, xs, f):
    return pl.pallas_call(
        _spmm_kernel,
        grid=(NEC,),
        in_specs=[_smem_spec(lambda i: (i, 0, 0)),
                  _smem_spec(lambda i: (i, 0, 0)),
                  _whole((N, f))],
        out_specs=_whole((N, f)),
        out_shape=jax.ShapeDtypeStruct((N, f), jnp.float32),
        interpret=_INTERPRET,
    )(row3, col3, xs)


def kernel(x_s, x_t, edge_index, emb, W1, b1, W2, b2, Wl, bl,
           L1W, L1b, L2W, L2b, L3W, L3b):
    f32 = jnp.float32
    idx = jnp.concatenate([x_s.reshape(-1, 11), x_t.reshape(-1, 34)],
                          axis=1).astype(jnp.int32)
    idxe = jnp.repeat(idx, 21, axis=1)
    ei = edge_index.astype(jnp.int32)
    row3 = jnp.concatenate([ei[0], ei[1]]).reshape(NEC, 1, ECHUNK)
    col3 = jnp.concatenate([ei[1], ei[0]]).reshape(NEC, 1, ECHUNK)

    eye45 = jnp.eye(45, dtype=f32)
    g2 = jnp.einsum('ab,vk->avbk', eye45, emb.astype(f32)).reshape(945, 450)
    g2 = jnp.pad(g2, ((0, 0), (0, F1 - 450)))
    bdw1 = jnp.einsum('ab,kc->akbc', eye45, W1.astype(f32)).reshape(450, 45 * 32)
    bdw1 = jnp.pad(bdw1, ((0, F1 - 450), (0, 0)))
    bdw2 = jnp.einsum('ab,kc->akbc', eye45, W2.astype(f32)).reshape(45 * 32, F2)
    b1t = jnp.tile(b1.astype(f32), 45).reshape(1, 45 * 32)
    b2t = jnp.tile(b2.astype(f32), 45).reshape(1, F2)

    dinv = pl.pallas_call(
        _deg_kernel,
        grid=(NEC,),
        in_specs=[_smem_spec(lambda i: (i, 0, 0))],
        out_specs=_whole((N, 1)),
        out_shape=jax.ShapeDtypeStruct((N, 1), f32),
        scratch_shapes=[pltpu.VMEM((N, 1), f32)],
        interpret=_INTERPRET,
    )(col3)

    xs = pl.pallas_call(
        _emb_kernel,
        grid=(NB,),
        in_specs=[pl.BlockSpec((B, 945), lambda i: (i, 0)),
                  pl.BlockSpec((B, 1), lambda i: (i, 0)),
                  _whole((945, F1))],
        out_specs=pl.BlockSpec((B, F1), lambda i: (i, 0)),
        out_shape=jax.ShapeDtypeStruct((N, F1), f32),
        interpret=_INTERPRET,
    )(idxe, dinv, g2)

    acc1 = _spmm(row3, col3, xs, F1)

    p = pl.pallas_call(
        _proj_kernel,
        grid=(NB,),
        in_specs=[pl.BlockSpec((B, F1), lambda i: (i, 0)),
                  pl.BlockSpec((B, 1), lambda i: (i, 0)),
                  _whole((F1, 45 * 32)),
                  _whole((1, 45 * 32)),
                  _whole((45 * 32, F2))],
        out_specs=pl.BlockSpec((B, F2), lambda i: (i, 0)),
        out_shape=jax.ShapeDtypeStruct((N, F2), f32),
        interpret=_INTERPRET,
    )(acc1, dinv, bdw1, b1t, bdw2)

    acc2 = _spmm(row3, col3, p, F2)

    out = pl.pallas_call(
        _mlp_kernel,
        grid=(NB,),
        in_specs=[pl.BlockSpec((B, F2), lambda i: (i, 0)),
                  pl.BlockSpec((B, 1), lambda i: (i, 0)),
                  _whole((1, F2)),
                  _whole((F2, 256)),
                  _whole((1, 256)),
                  _whole((256, 256)),
                  _whole((1, 256)),
                  _whole((256, 128)),
                  _whole((1, 128)),
                  _whole((128, 1)),
                  _whole((1, 1))],
        out_specs=pl.BlockSpec((B, 1), lambda i: (i, 0)),
        out_shape=jax.ShapeDtypeStruct((N, 1), f32),
        interpret=_INTERPRET,
    )(acc2, dinv, b2t, Wl.astype(f32), bl.reshape(1, -1).astype(f32),
      L1W.astype(f32), L1b.reshape(1, -1).astype(f32),
      L2W.astype(f32), L2b.reshape(1, -1).astype(f32),
      L3W.astype(f32), L3b.reshape(1, -1).astype(f32))
    return out


# speedup vs baseline: 15.1773x; 1.2825x over previous
import jax
import jax.numpy as jnp
from jax import lax
from jax.experimental import pallas as pl
from jax.experimental.pallas import tpu as pltpu
from jax.experimental.pallas import tpu_sc as plsc

N = 10000
NPAD = 10112
E = 30000
NC = 2
NS = 16
NW = NC * NS
EPW = 960
EPAD = NW * EPW
G = 96
NG = EPW // G
RPT = NPAD // NS
B = 1000
NB = N // B
F1 = 512
C1 = 128
F2 = 768
C2 = 128
NCH2 = F2 // C2

_SC_MESH = dict(core_axis_name="c", subcore_axis_name="s")


def _sc_spmm(rowg, colg, xs_chunks):
    nch = len(xs_chunks)
    C = xs_chunks[0].shape[1]
    nz, rz = RPT // 16, RPT % 16

    def body(rowg_ref, colg_ref, *rest):
        xs = rest[:nch]
        outs = rest[nch:2 * nch]
        rowt, colt, buf0, buf1, zbuf, acc, sem = rest[2 * nch:]
        cid = lax.axis_index("c")
        sid = lax.axis_index("s")
        wid = cid * NS + sid
        pltpu.sync_copy(rowg_ref.at[wid], rowt)
        pltpu.sync_copy(colg_ref.at[wid], colt)
        for i in range(16):
            for j in range(C // 16):
                zbuf[i, pl.ds(j * 16, 16)] = jnp.zeros((16,), jnp.float32)
        base = sid * RPT
        bufs = (buf0, buf1)
        for k in range(nch):
            def z(r, c):
                pltpu.sync_copy(zbuf, acc.at[pl.ds(base + r * 16, 16)])
                return c

            lax.fori_loop(0, nz, z, 0)
            if rz:
                pltpu.sync_copy(zbuf.at[pl.ds(0, rz)],
                                acc.at[pl.ds(base + nz * 16, rz)])
            plsc.subcore_barrier()
            d = pltpu.async_copy(xs[k].at[rowt.at[0]], buf0, sem)
            for g in range(NG):
                d.wait()
                if g + 1 < NG:
                    d = pltpu.async_copy(xs[k].at[rowt.at[g + 1]],
                                         bufs[(g + 1) % 2], sem)
                pltpu.sync_copy(bufs[g % 2], acc.at[colt.at[g]], add=True)
            plsc.subcore_barrier()
            pltpu.sync_copy(acc.at[pl.ds(base, RPT)],
                            outs[k].at[cid, pl.ds(base, RPT)])

    kern = pl.kernel(
        body,
        out_type=[jax.ShapeDtypeStruct((NC, NPAD, C), jnp.float32)] * nch,
        mesh=plsc.VectorSubcoreMesh(**_SC_MESH),
        scratch_types=[
            pltpu.VMEM((NG, G), jnp.int32),
            pltpu.VMEM((NG, G), jnp.int32),
            pltpu.VMEM((G, C), jnp.float32),
            pltpu.VMEM((G, C), jnp.float32),
            pltpu.VMEM((16, C), jnp.float32),
            pltpu.VMEM_SHARED((NPAD, C), jnp.float32),
            pltpu.SemaphoreType.DMA,
        ],
    )
    outs = kern(rowg, colg, *xs_chunks)
    return list(outs) if isinstance(outs, (list, tuple)) else [outs]


def _emb_kernel(idxe_ref, hist_ref, g2_ref, dinv_ref, *xs_refs):
    lane = jax.lax.broadcasted_iota(jnp.int32, (B, 45 * 21), 1) % 21
    u = (idxe_ref[...] == lane).astype(jnp.float32)
    x = jnp.dot(u, g2_ref[...], preferred_element_type=jnp.float32)
    sel = (jax.lax.broadcasted_iota(jnp.int32, (C1, 1), 0) == 0
           ).astype(jnp.float32)
    h = hist_ref[0] + hist_ref[1]
    s = jnp.dot(h, sel, preferred_element_type=jnp.float32)
    d = lax.rsqrt(1.0 + s)
    dinv_ref[...] = d
    xsc = x * d
    for k, r in enumerate(xs_refs):
        r[...] = xsc[:, k * C1:(k + 1) * C1]


def _proj_kernel(dinv_ref, *rest):
    xs = rest[:4]
    p = rest[4:8]
    w1_ref, b1_ref, w2_ref = rest[8:11]
    p_out = rest[11:]
    d = dinv_ref[...]
    a = jnp.concatenate(
        [xs[k][...] + p[k][0] + p[k][1] for k in range(4)], axis=1)
    h = jnp.dot(a * d, w1_ref[...],
                preferred_element_type=jnp.float32) + b1_ref[...]
    h = jnp.maximum(h, 0.0)
    pr = jnp.dot(h, w2_ref[...], preferred_element_type=jnp.float32) * d
    for k, r in enumerate(p_out):
        r[...] = pr[:, k * C2:(k + 1) * C2]


def _mlp_kernel(dinv_ref, *rest):
    p = rest[:NCH2]
    q = rest[NCH2:2 * NCH2]
    (b2_ref, wl_ref, bl_ref, l1w_ref, l1b_ref, l2w_ref, l2b_ref,
     l3w_ref, l3b_ref, out_ref) = rest[2 * NCH2:]
    d = dinv_ref[...]
    flat = jnp.concatenate(
        [p[k][...] + q[k][0] + q[k][1] for k in range(NCH2)], axis=1)
    flat = flat * d + b2_ref[...]
    o = jnp.dot(flat, wl_ref[...],
                preferred_element_type=jnp.float32) + bl_ref[...]
    o = jnp.where(o >= 0.0, o, 0.01 * o)
    o = jnp.maximum(jnp.dot(o, l1w_ref[...],
                            preferred_element_type=jnp.float32) + l1b_ref[...], 0.0)
    o = jnp.maximum(jnp.dot(o, l2w_ref[...],
                            preferred_element_type=jnp.float32) + l2b_ref[...], 0.0)
    z = jnp.dot(o, l3w_ref[...], preferred_element_type=jnp.float32) + l3b_ref[...]
    out_ref[...] = 1.0 / (1.0 + jnp.exp(-z))


def _whole(shape):
    return pl.BlockSpec(shape, lambda i: (0,) * len(shape))


def _part_spec(C):
    return pl.BlockSpec((NC, B, C), lambda i: (0, i, 0))


def kernel(x_s, x_t, edge_index, emb, W1, b1, W2, b2, Wl, bl,
           L1W, L1b, L2W, L2b, L3W, L3b):
    f32 = jnp.float32
    idx = jnp.concatenate([x_s.reshape(-1, 11), x_t.reshape(-1, 34)],
                          axis=1).astype(jnp.int32)
    idxe = jnp.repeat(idx, 21, axis=1)
    ei = edge_index.astype(jnp.int32)
    pad = jnp.full((EPAD - 2 * (E // 2),), N, jnp.int32)
    rows_all = jnp.concatenate([ei[0], ei[1], pad])
    cols_all = jnp.concatenate([ei[1], ei[0], pad])
    rowg = rows_all.reshape(NW, NG, G)
    colg = cols_all.reshape(NW, NG, G)

    eye45 = jnp.eye(45, dtype=f32)
    g2 = jnp.einsum('ab,vk->avbk', eye45, emb.astype(f32)).reshape(945, 450)
    g2 = jnp.pad(g2, ((0, 0), (0, F1 - 450)))
    bdw1 = jnp.einsum('ab,kc->akbc', eye45, W1.astype(f32)).reshape(450, 45 * 32)
    bdw1 = jnp.pad(bdw1, ((0, F1 - 450), (0, 0)))
    bdw2 = jnp.einsum('ab,kc->akbc', eye45, W2.astype(f32)).reshape(45 * 32, 720)
    bdw2 = jnp.pad(bdw2, ((0, 0), (0, F2 - 720)))
    b1t = jnp.tile(b1.astype(f32), 45).reshape(1, 45 * 32)
    b2t = jnp.pad(jnp.tile(b2.astype(f32), 45).reshape(1, 720),
                  ((0, 0), (0, F2 - 720)))
    wl_pad = jnp.pad(Wl.astype(f32), ((0, F2 - 720), (0, 0)))

    deg_parts = _sc_spmm(rowg, colg, [jnp.ones((NPAD, C1), f32)])[0]

    dinv, xs0, xs1, xs2, xs3 = pl.pallas_call(
        _emb_kernel,
        grid=(NB,),
        in_specs=[pl.BlockSpec((B, 945), lambda i: (i, 0)),
                  _part_spec(C1),
                  _whole((945, F1))],
        out_specs=[pl.BlockSpec((B, 1), lambda i: (i, 0))] +
                  [pl.BlockSpec((B, C1), lambda i: (i, 0))] * 4,
        out_shape=[jax.ShapeDtypeStruct((N, 1), f32)] +
                  [jax.ShapeDtypeStruct((NPAD, C1), f32)] * 4,
    )(idxe, deg_parts, g2)

    p_parts = _sc_spmm(rowg, colg, [xs0, xs1, xs2, xs3])

    prs = pl.pallas_call(
        _proj_kernel,
        grid=(NB,),
        in_specs=[pl.BlockSpec((B, 1), lambda i: (i, 0))] +
                 [pl.BlockSpec((B, C1), lambda i: (i, 0))] * 4 +
                 [_part_spec(C1)] * 4 +
                 [_whole((F1, 45 * 32)),
                  _whole((1, 45 * 32)),
                  _whole((45 * 32, F2))],
        out_specs=[pl.BlockSpec((B, C2), lambda i: (i, 0))] * NCH2,
        out_shape=[jax.ShapeDtypeStruct((NPAD, C2), f32)] * NCH2,
    )(dinv, xs0, xs1, xs2, xs3, *p_parts, bdw1, b1t, bdw2)

    q_parts = _sc_spmm(rowg, colg, list(prs))

    out = pl.pallas_call(
        _mlp_kernel,
        grid=(NB,),
        in_specs=[pl.BlockSpec((B, 1), lambda i: (i, 0))] +
                 [pl.BlockSpec((B, C2), lambda i: (i, 0))] * NCH2 +
                 [_part_spec(C2)] * NCH2 +
                 [_whole((1, F2)),
                  _whole((F2, 256)),
                  _whole((1, 256)),
                  _whole((256, 256)),
                  _whole((1, 256)),
                  _whole((256, 128)),
                  _whole((1, 128)),
                  _whole((128, 1)),
                  _whole((1, 1))],
        out_specs=pl.BlockSpec((B, 1), lambda i: (i, 0)),
        out_shape=jax.ShapeDtypeStruct((N, 1), f32),
    )(dinv, *prs, *q_parts, b2t, wl_pad, bl.reshape(1, -1).astype(f32),
      L1W.astype(f32), L1b.reshape(1, -1).astype(f32),
      L2W.astype(f32), L2b.reshape(1, -1).astype(f32),
      L3W.astype(f32), L3b.reshape(1, -1).astype(f32))
    return out


# speedup vs baseline: 21.4674x; 1.4144x over previous
import jax
import jax.numpy as jnp
from jax import lax
from jax.experimental import pallas as pl
from jax.experimental.pallas import tpu as pltpu
from jax.experimental.pallas import tpu_sc as plsc

N = 10000
NPAD = 10112
E = 30000
NC = 2
NS = 16
NW = NC * NS
EPW = 960
EPAD = NW * EPW
G = 96
NG = EPW // G
RPT = NPAD // NS
B = 1000
NB = N // B
F1 = 512
C1 = 128
F2 = 256
C2 = 128
NCH2 = F2 // C2

_SC_MESH = dict(core_axis_name="c", subcore_axis_name="s")


def _sc_spmm(rowg, colg, xs_chunks):
    nch = len(xs_chunks)
    C = xs_chunks[0].shape[1]
    nz, rz = RPT // 16, RPT % 16

    def body(rowg_ref, colg_ref, *rest):
        xs = rest[:nch]
        outs = rest[nch:2 * nch]
        rowt, colt, buf0, buf1, zbuf, acc, sem = rest[2 * nch:]
        cid = lax.axis_index("c")
        sid = lax.axis_index("s")
        wid = cid * NS + sid
        pltpu.sync_copy(rowg_ref.at[wid], rowt)
        pltpu.sync_copy(colg_ref.at[wid], colt)
        for i in range(16):
            for j in range(C // 16):
                zbuf[i, pl.ds(j * 16, 16)] = jnp.zeros((16,), jnp.float32)
        base = sid * RPT
        bufs = (buf0, buf1)
        for k in range(nch):
            def z(r, c):
                pltpu.sync_copy(zbuf, acc.at[pl.ds(base + r * 16, 16)])
                return c

            lax.fori_loop(0, nz, z, 0)
            if rz:
                pltpu.sync_copy(zbuf.at[pl.ds(0, rz)],
                                acc.at[pl.ds(base + nz * 16, rz)])
            plsc.subcore_barrier()
            d = pltpu.async_copy(xs[k].at[rowt.at[0]], buf0, sem)
            for g in range(NG):
                d.wait()
                if g + 1 < NG:
                    d = pltpu.async_copy(xs[k].at[rowt.at[g + 1]],
                                         bufs[(g + 1) % 2], sem)
                pltpu.sync_copy(bufs[g % 2], acc.at[colt.at[g]], add=True)
            plsc.subcore_barrier()
            pltpu.sync_copy(acc.at[pl.ds(base, RPT)],
                            outs[k].at[cid, pl.ds(base, RPT)])

    kern = pl.kernel(
        body,
        out_type=[jax.ShapeDtypeStruct((NC, NPAD, C), jnp.float32)] * nch,
        mesh=plsc.VectorSubcoreMesh(**_SC_MESH),
        scratch_types=[
            pltpu.VMEM((NG, G), jnp.int32),
            pltpu.VMEM((NG, G), jnp.int32),
            pltpu.VMEM((G, C), jnp.float32),
            pltpu.VMEM((G, C), jnp.float32),
            pltpu.VMEM((16, C), jnp.float32),
            pltpu.VMEM_SHARED((NPAD, C), jnp.float32),
            pltpu.SemaphoreType.DMA,
        ],
    )
    outs = kern(rowg, colg, *xs_chunks)
    return list(outs) if isinstance(outs, (list, tuple)) else [outs]


def _emb_kernel(idxe_ref, hist_ref, g2_ref, dinv_ref, *xs_refs):
    lane = jax.lax.broadcasted_iota(jnp.int32, (B, 45 * 21), 1) % 21
    u = (idxe_ref[...] == lane).astype(jnp.float32)
    x = jnp.dot(u, g2_ref[...], preferred_element_type=jnp.float32)
    sel = (jax.lax.broadcasted_iota(jnp.int32, (C1, 1), 0) == 0
           ).astype(jnp.float32)
    h = hist_ref[0] + hist_ref[1]
    s = jnp.dot(h, sel, preferred_element_type=jnp.float32)
    d = lax.rsqrt(1.0 + s)
    dinv_ref[...] = d
    xsc = x * d
    for k, r in enumerate(xs_refs):
        r[...] = xsc[:, k * C1:(k + 1) * C1]


def _fold_kernel(bdw2_ref, wl_ref, b2t_ref, bl_ref, w2l_ref, blp_ref):
    w2l_ref[...] = jnp.dot(bdw2_ref[...], wl_ref[...],
                           preferred_element_type=jnp.float32)
    blp_ref[...] = jnp.dot(b2t_ref[...], wl_ref[...],
                           preferred_element_type=jnp.float32) + bl_ref[...]


def _proj_kernel(dinv_ref, *rest):
    xs = rest[:4]
    p = rest[4:8]
    w1_ref, b1_ref, w2_ref = rest[8:11]
    p_out = rest[11:]
    d = dinv_ref[...]
    a = jnp.concatenate(
        [xs[k][...] + p[k][0] + p[k][1] for k in range(4)], axis=1)
    h = jnp.dot(a * d, w1_ref[...],
                preferred_element_type=jnp.float32) + b1_ref[...]
    h = jnp.maximum(h, 0.0)
    pr = jnp.dot(h, w2_ref[...], preferred_element_type=jnp.float32) * d
    for k, r in enumerate(p_out):
        r[...] = pr[:, k * C2:(k + 1) * C2]


def _mlp_kernel(dinv_ref, *rest):
    p = rest[:NCH2]
    q = rest[NCH2:2 * NCH2]
    (blp_ref, l1w_ref, l1b_ref, l2w_ref, l2b_ref,
     l3w_ref, l3b_ref, out_ref) = rest[2 * NCH2:]
    d = dinv_ref[...]
    flat = jnp.concatenate(
        [p[k][...] + q[k][0] + q[k][1] for k in range(NCH2)], axis=1)
    o = flat * d + blp_ref[...]
    o = jnp.where(o >= 0.0, o, 0.01 * o)
    o = jnp.maximum(jnp.dot(o, l1w_ref[...],
                            preferred_element_type=jnp.float32) + l1b_ref[...], 0.0)
    o = jnp.maximum(jnp.dot(o, l2w_ref[...],
                            preferred_element_type=jnp.float32) + l2b_ref[...], 0.0)
    z = jnp.dot(o, l3w_ref[...], preferred_element_type=jnp.float32) + l3b_ref[...]
    out_ref[...] = 1.0 / (1.0 + jnp.exp(-z))


def _whole(shape):
    return pl.BlockSpec(shape, lambda i: (0,) * len(shape))


def _part_spec(C):
    return pl.BlockSpec((NC, B, C), lambda i: (0, i, 0))


def kernel(x_s, x_t, edge_index, emb, W1, b1, W2, b2, Wl, bl,
           L1W, L1b, L2W, L2b, L3W, L3b):
    f32 = jnp.float32
    idx = jnp.concatenate([x_s.reshape(-1, 11), x_t.reshape(-1, 34)],
                          axis=1).astype(jnp.int32)
    idxe = jnp.repeat(idx, 21, axis=1)
    ei = edge_index.astype(jnp.int32)
    pad = jnp.full((EPAD - 2 * (E // 2),), N, jnp.int32)
    rows_all = jnp.concatenate([ei[0], ei[1], pad])
    cols_all = jnp.concatenate([ei[1], ei[0], pad])
    rowg = rows_all.reshape(NW, NG, G)
    colg = cols_all.reshape(NW, NG, G)

    eye45 = jnp.eye(45, dtype=f32)
    g2 = jnp.einsum('ab,vk->avbk', eye45, emb.astype(f32)).reshape(945, 450)
    g2 = jnp.pad(g2, ((0, 0), (0, F1 - 450)))
    bdw1 = jnp.einsum('ab,kc->akbc', eye45, W1.astype(f32)).reshape(450, 45 * 32)
    bdw1 = jnp.pad(bdw1, ((0, F1 - 450), (0, 0)))
    bdw2 = jnp.einsum('ab,kc->akbc', eye45, W2.astype(f32)).reshape(45 * 32, 720)
    b1t = jnp.tile(b1.astype(f32), 45).reshape(1, 45 * 32)
    b2t = jnp.tile(b2.astype(f32), 45).reshape(1, 720)

    w2l, blp = pl.pallas_call(
        _fold_kernel,
        grid=(1,),
        in_specs=[_whole((45 * 32, 720)), _whole((720, F2)),
                  _whole((1, 720)), _whole((1, F2))],
        out_specs=[_whole((45 * 32, F2)), _whole((1, F2))],
        out_shape=[jax.ShapeDtypeStruct((45 * 32, F2), f32),
                   jax.ShapeDtypeStruct((1, F2), f32)],
    )(bdw2, Wl.astype(f32), b2t, bl.reshape(1, -1).astype(f32))

    deg_parts = _sc_spmm(rowg, colg, [jnp.ones((NPAD, C1), f32)])[0]

    dinv, xs0, xs1, xs2, xs3 = pl.pallas_call(
        _emb_kernel,
        grid=(NB,),
        in_specs=[pl.BlockSpec((B, 945), lambda i: (i, 0)),
                  _part_spec(C1),
                  _whole((945, F1))],
        out_specs=[pl.BlockSpec((B, 1), lambda i: (i, 0))] +
                  [pl.BlockSpec((B, C1), lambda i: (i, 0))] * 4,
        out_shape=[jax.ShapeDtypeStruct((N, 1), f32)] +
                  [jax.ShapeDtypeStruct((NPAD, C1), f32)] * 4,
    )(idxe, deg_parts, g2)

    p_parts = _sc_spmm(rowg, colg, [xs0, xs1, xs2, xs3])

    prs = pl.pallas_call(
        _proj_kernel,
        grid=(NB,),
        in_specs=[pl.BlockSpec((B, 1), lambda i: (i, 0))] +
                 [pl.BlockSpec((B, C1), lambda i: (i, 0))] * 4 +
                 [_part_spec(C1)] * 4 +
                 [_whole((F1, 45 * 32)),
                  _whole((1, 45 * 32)),
                  _whole((45 * 32, F2))],
        out_specs=[pl.BlockSpec((B, C2), lambda i: (i, 0))] * NCH2,
        out_shape=[jax.ShapeDtypeStruct((NPAD, C2), f32)] * NCH2,
    )(dinv, xs0, xs1, xs2, xs3, *p_parts, bdw1, b1t, w2l)

    q_parts = _sc_spmm(rowg, colg, list(prs))

    out = pl.pallas_call(
        _mlp_kernel,
        grid=(NB,),
        in_specs=[pl.BlockSpec((B, 1), lambda i: (i, 0))] +
                 [pl.BlockSpec((B, C2), lambda i: (i, 0))] * NCH2 +
                 [_part_spec(C2)] * NCH2 +
                 [_whole((1, F2)),
                  _whole((256, 256)),
                  _whole((1, 256)),
                  _whole((256, 128)),
                  _whole((1, 128)),
                  _whole((128, 1)),
                  _whole((1, 1))],
        out_specs=pl.BlockSpec((B, 1), lambda i: (i, 0)),
        out_shape=jax.ShapeDtypeStruct((N, 1), f32),
    )(dinv, *prs, *q_parts, blp,
      L1W.astype(f32), L1b.reshape(1, -1).astype(f32),
      L2W.astype(f32), L2b.reshape(1, -1).astype(f32),
      L3W.astype(f32), L3b.reshape(1, -1).astype(f32))
    return out
